# Initial kernel scaffold; baseline (speedup 1.0000x reference)
import jax
import jax.numpy as jnp
from jax.experimental import pallas as pl


def kernel(query, all_coords, scale_ranges, reference_points, input_flatten, W_offsets, b_offsets, W_attn, b_attn, W_value, b_value, W_out, b_out):
    raise NotImplementedError("write your pallas kernel here")


# speedup vs baseline: 15.2706x; 15.2706x over previous
import jax
import jax.numpy as jnp
from jax.experimental import pallas as pl
from jax.experimental.pallas import tpu as pltpu

D_MODEL = 512
N_HEADS = 26
N_POINTS = 4
K_ACT = 8
D_HEAD = 64
KNN = 3
NEG = -3.4e38
INF = 3.4e38


def _proj_kernel(q_ref, Wa_ref, ba_ref, Wo_ref, bo_ref, x_ref, Wv_ref, bv_ref,
                 attn_ref, offs_ref, val_ref, idx_ref):
    f32 = jnp.float32
    q = q_ref[...]
    attn = jnp.dot(q, Wa_ref[...], preferred_element_type=f32) + ba_ref[...]
    attn_ref[...] = attn
    offs_ref[...] = jnp.dot(q, Wo_ref[...], preferred_element_type=f32) + bo_ref[...]
    val_ref[...] = jnp.dot(x_ref[...], Wv_ref[...], preferred_element_type=f32) + bv_ref[...]

    rows = jax.lax.broadcasted_iota(jnp.int32, (N_HEADS * N_POINTS, N_HEADS), 0)
    cols = jax.lax.broadcasted_iota(jnp.int32, (N_HEADS * N_POINTS, N_HEADS), 1)
    S = (rows // N_POINTS == cols).astype(f32)
    scores = jnp.dot(attn, S, preferred_element_type=f32, precision=jax.lax.Precision.HIGHEST)
    iota26 = jax.lax.broadcasted_iota(jnp.int32, scores.shape, 1)
    for kk in range(K_ACT):
        m = jnp.max(scores, axis=1, keepdims=True)
        idx = jnp.min(jnp.where(scores == m, iota26, N_HEADS), axis=1,
                      keepdims=True)
        idx_ref[kk] = idx
        scores = jnp.where(iota26 == idx, NEG, scores)


def _gsel(n_cols, width):
    r = jax.lax.broadcasted_iota(jnp.int32, (n_cols, width), 0)
    c = jax.lax.broadcasted_iota(jnp.int32, (n_cols, width), 1)
    return (r % width == c).astype(jnp.float32)


def _level_kernel(idx_ref, attn_ref, offs_ref, val_ref, ref12_ref, smin12_ref,
                  den12_ref, nsrcT_ref, Wout_ref, bout_ref, out_ref):
    k = pl.program_id(0)
    f32 = jnp.float32
    Lq = attn_ref.shape[0]
    Nn = nsrcT_ref.shape[1]

    @pl.when(k == 0)
    def _init():
        out_ref[...] = jnp.zeros((Lq, D_MODEL), f32) + bout_ref[...]

    idxc = idx_ref[0]

    def _select(ref, width):
        n_cols = ref.shape[1]
        heads = jax.lax.broadcasted_iota(jnp.int32, (Lq, n_cols), 1) // width
        sel = jnp.where(heads == idxc, ref[...], 0.0)
        return jnp.dot(sel, _gsel(n_cols, width), preferred_element_type=f32, precision=jax.lax.Precision.HIGHEST)

    attn4 = _select(attn_ref, N_POINTS)
    samp12 = _select(offs_ref, 12)
    lvl_val = _select(val_ref, D_HEAD)

    amax = jnp.max(attn4, axis=1, keepdims=True)
    ae = jnp.exp(attn4 - amax)
    aw4 = ae / jnp.sum(ae, axis=1, keepdims=True)
    nloc12 = ((ref12_ref[...] + samp12) - smin12_ref[...]) / den12_ref[...]

    nsrc3 = nsrcT_ref[...]
    s0 = nsrc3[0:1, :]
    s1 = nsrc3[1:2, :]
    s2 = nsrc3[2:3, :]
    ss = s0 * s0 + s1 * s1 + s2 * s2

    iota_m = jax.lax.broadcasted_iota(jnp.int32, (Lq, Nn), 1)
    Wmat = jnp.zeros((Lq, Nn), f32)
    for p in range(N_POINTS):
        nl = nloc12[:, p * 3:p * 3 + 3]
        a0 = nl[:, 0:1]
        a1 = nl[:, 1:2]
        a2 = nl[:, 2:3]
        qq = a0 * a0 + a1 * a1 + a2 * a2
        cross = jnp.dot(nl, nsrc3, preferred_element_type=f32)
        sq = (qq + ss) - 2.0 * cross
        dist = jnp.sqrt(jnp.maximum(sq, 1e-12))
        ms, idxs = [], []
        dcur = dist
        for j in range(KNN):
            m = jnp.min(dcur, axis=1, keepdims=True)
            i = jnp.min(jnp.where(dcur == m, iota_m, Nn), axis=1, keepdims=True)
            ms.append(m)
            idxs.append(i)
            if j < KNN - 1:
                dcur = jnp.where(iota_m == i, INF, dcur)
        us = [1.0 / (m + 1e-7) for m in ms]
        usum = us[0] + us[1] + us[2]
        awp = aw4[:, p:p + 1]
        for j in range(KNN):
            Wmat = Wmat + ((awp * us[j]) / usum) * (iota_m == idxs[j]).astype(f32)

    out_lvl = jnp.dot(Wmat, lvl_val, preferred_element_type=f32, precision=jax.lax.Precision.HIGHEST)
    wout_k = Wout_ref[pl.ds(k * D_HEAD, D_HEAD), :]
    out_ref[...] += jnp.dot(out_lvl, wout_k, preferred_element_type=f32)


def _full(arr):
    nd = arr.ndim
    return pl.BlockSpec(arr.shape, lambda k, _n=nd: (0,) * _n)


@jax.jit
def kernel(query, all_coords, scale_ranges, reference_points, input_flatten,
           W_offsets, b_offsets, W_attn, b_attn, W_value, b_value, W_out,
           b_out):
    B, Lq, _ = query.shape
    Nn = input_flatten.shape[1]
    q = query[0]
    x = input_flatten[0]
    smin = scale_ranges[0, 0, :]
    denom = scale_ranges[0, 1, :] - smin + 1e-7
    ref12 = jnp.tile(reference_points[0], (1, N_POINTS))
    smin12 = jnp.tile(smin[None, :], (1, N_POINTS))
    den12 = jnp.tile(denom[None, :], (1, N_POINTS))
    nsrcT = ((all_coords[0] - smin[None, :]) / denom[None, :]).T

    attn, offs, val, idx8 = pl.pallas_call(
        _proj_kernel,
        out_shape=[
            jax.ShapeDtypeStruct((Lq, N_HEADS * N_POINTS), jnp.float32),
            jax.ShapeDtypeStruct((Lq, N_HEADS * 12), jnp.float32),
            jax.ShapeDtypeStruct((Lq, N_HEADS * D_HEAD), jnp.float32),
            jax.ShapeDtypeStruct((K_ACT, Lq, 1), jnp.int32),
        ],
    )(q, W_attn, b_attn[None, :], W_offsets, b_offsets[None, :],
      x, W_value, b_value[None, :])

    out = pl.pallas_call(
        _level_kernel,
        grid=(K_ACT,),
        in_specs=[
            pl.BlockSpec((1, Lq, 1), lambda k: (k, 0, 0)),
            _full(attn),
            _full(offs),
            _full(val),
            _full(ref12),
            _full(smin12),
            _full(den12),
            _full(nsrcT),
            _full(W_out),
            pl.BlockSpec((1, D_MODEL), lambda k: (0, 0)),
        ],
        out_specs=pl.BlockSpec((Lq, D_MODEL), lambda k: (0, 0)),
        out_shape=jax.ShapeDtypeStruct((Lq, D_MODEL), jnp.float32),
    )(idx8, attn, offs, val, ref12, smin12, den12, nsrcT, W_out, b_out[None, :])
    return out[None]


# speedup vs baseline: 20.9171x; 1.3698x over previous
import jax
import jax.numpy as jnp
from jax.experimental import pallas as pl
from jax.experimental.pallas import tpu as pltpu

D_MODEL = 512
N_HEADS = 26
N_POINTS = 4
K_ACT = 8
D_HEAD = 64
KNN = 3
NEG = -3.4e38
INF = 3.4e38


def _proj_kernel(q_ref, Wa_ref, ba_ref, Wo_ref, bo_ref, x_ref, Wv_ref, bv_ref,
                 attn_ref, offs_ref, val_ref, idx_ref):
    f32 = jnp.float32
    q = q_ref[...]
    attn = jnp.dot(q, Wa_ref[...], preferred_element_type=f32) + ba_ref[...]
    attn_ref[...] = attn
    offs_ref[...] = jnp.dot(q, Wo_ref[...], preferred_element_type=f32) + bo_ref[...]
    val_ref[...] = jnp.dot(x_ref[...], Wv_ref[...], preferred_element_type=f32) + bv_ref[...]

    rows = jax.lax.broadcasted_iota(jnp.int32, (N_HEADS * N_POINTS, N_HEADS), 0)
    cols = jax.lax.broadcasted_iota(jnp.int32, (N_HEADS * N_POINTS, N_HEADS), 1)
    S = (rows // N_POINTS == cols).astype(f32)
    scores = jnp.dot(attn, S, preferred_element_type=f32, precision=jax.lax.Precision.HIGHEST)
    iota26 = jax.lax.broadcasted_iota(jnp.int32, scores.shape, 1)
    for kk in range(K_ACT):
        m = jnp.max(scores, axis=1, keepdims=True)
        idx = jnp.min(jnp.where(scores == m, iota26, N_HEADS), axis=1,
                      keepdims=True)
        idx_ref[kk] = idx
        scores = jnp.where(iota26 == idx, NEG, scores)


def _gsel(n_cols, width):
    r = jax.lax.broadcasted_iota(jnp.int32, (n_cols, width), 0)
    c = jax.lax.broadcasted_iota(jnp.int32, (n_cols, width), 1)
    return (r % width == c).astype(jnp.float32)


def _level_kernel(idx_ref, attn_ref, offs_ref, val_ref, ref12_ref, smin12_ref,
                  den12_ref, nsrcT_ref, Wout_ref, bout_ref, out_ref):
    k = pl.program_id(0)
    f32 = jnp.float32
    Lq = attn_ref.shape[0]
    Nn = nsrcT_ref.shape[1]

    @pl.when(k == 0)
    def _init():
        out_ref[...] = jnp.zeros((Lq, D_MODEL), f32) + bout_ref[...]

    idxc = idx_ref[0]

    def _select(ref, width):
        n_cols = ref.shape[1]
        heads = jax.lax.broadcasted_iota(jnp.int32, (Lq, n_cols), 1) // width
        sel = jnp.where(heads == idxc, ref[...], 0.0)
        return jnp.dot(sel, _gsel(n_cols, width), preferred_element_type=f32, precision=jax.lax.Precision.HIGHEST)

    attn4 = _select(attn_ref, N_POINTS)
    samp12 = _select(offs_ref, 12)
    lvl_val = _select(val_ref, D_HEAD)

    amax = jnp.max(attn4, axis=1, keepdims=True)
    ae = jnp.exp(attn4 - amax)
    aw4 = ae / jnp.sum(ae, axis=1, keepdims=True)
    nloc12 = ((ref12_ref[...] + samp12) - smin12_ref[...]) / den12_ref[...]

    nsrc3 = nsrcT_ref[...]
    s0 = nsrc3[0:1, :]
    s1 = nsrc3[1:2, :]
    s2 = nsrc3[2:3, :]
    ss = s0 * s0 + s1 * s1 + s2 * s2

    iota_m = jax.lax.broadcasted_iota(jnp.int32, (Lq, Nn), 1)
    Wmat = jnp.zeros((Lq, Nn), f32)
    for p in range(N_POINTS):
        nl = nloc12[:, p * 3:p * 3 + 3]
        a0 = nl[:, 0:1]
        a1 = nl[:, 1:2]
        a2 = nl[:, 2:3]
        qq = a0 * a0 + a1 * a1 + a2 * a2
        cross = jnp.dot(nl, nsrc3, preferred_element_type=f32)
        sq = (qq + ss) - 2.0 * cross
        ms, ohs = [], []
        dcur = jnp.maximum(sq, 1e-12)
        for j in range(KNN):
            m = jnp.min(dcur, axis=1, keepdims=True)
            i = jnp.min(jnp.where(dcur == m, iota_m, Nn), axis=1, keepdims=True)
            oh = iota_m == i
            ms.append(m)
            ohs.append(oh)
            if j < KNN - 1:
                dcur = jnp.where(oh, INF, dcur)
        us = [1.0 / (jnp.sqrt(m) + 1e-7) for m in ms]
        usum = us[0] + us[1] + us[2]
        awp = aw4[:, p:p + 1]
        cs = [(awp * u) / usum for u in us]
        Wmat = Wmat + jnp.where(
            ohs[0], cs[0], jnp.where(ohs[1], cs[1],
                                     jnp.where(ohs[2], cs[2], 0.0)))

    out_lvl = jnp.dot(Wmat, lvl_val, preferred_element_type=f32, precision=jax.lax.Precision.HIGHEST)
    wout_k = Wout_ref[pl.ds(k * D_HEAD, D_HEAD), :]
    out_ref[...] += jnp.dot(out_lvl, wout_k, preferred_element_type=f32)


def _full(arr):
    nd = arr.ndim
    return pl.BlockSpec(arr.shape, lambda k, _n=nd: (0,) * _n)


@jax.jit
def kernel(query, all_coords, scale_ranges, reference_points, input_flatten,
           W_offsets, b_offsets, W_attn, b_attn, W_value, b_value, W_out,
           b_out):
    B, Lq, _ = query.shape
    Nn = input_flatten.shape[1]
    q = query[0]
    x = input_flatten[0]
    smin = scale_ranges[0, 0, :]
    denom = scale_ranges[0, 1, :] - smin + 1e-7
    ref12 = jnp.tile(reference_points[0], (1, N_POINTS))
    smin12 = jnp.tile(smin[None, :], (1, N_POINTS))
    den12 = jnp.tile(denom[None, :], (1, N_POINTS))
    nsrcT = ((all_coords[0] - smin[None, :]) / denom[None, :]).T

    attn, offs, val, idx8 = pl.pallas_call(
        _proj_kernel,
        out_shape=[
            jax.ShapeDtypeStruct((Lq, N_HEADS * N_POINTS), jnp.float32),
            jax.ShapeDtypeStruct((Lq, N_HEADS * 12), jnp.float32),
            jax.ShapeDtypeStruct((Lq, N_HEADS * D_HEAD), jnp.float32),
            jax.ShapeDtypeStruct((K_ACT, Lq, 1), jnp.int32),
        ],
    )(q, W_attn, b_attn[None, :], W_offsets, b_offsets[None, :],
      x, W_value, b_value[None, :])

    out = pl.pallas_call(
        _level_kernel,
        grid=(K_ACT,),
        in_specs=[
            pl.BlockSpec((1, Lq, 1), lambda k: (k, 0, 0)),
            _full(attn),
            _full(offs),
            _full(val),
            _full(ref12),
            _full(smin12),
            _full(den12),
            _full(nsrcT),
            _full(W_out),
            pl.BlockSpec((1, D_MODEL), lambda k: (0, 0)),
        ],
        out_specs=pl.BlockSpec((Lq, D_MODEL), lambda k: (0, 0)),
        out_shape=jax.ShapeDtypeStruct((Lq, D_MODEL), jnp.float32),
    )(idx8, attn, offs, val, ref12, smin12, den12, nsrcT, W_out, b_out[None, :])
    return out[None]
